# Initial kernel scaffold; baseline (speedup 1.0000x reference)
import jax
import jax.numpy as jnp
from jax.experimental import pallas as pl


def kernel(positions, node_features, global_features, params):
    raise NotImplementedError("write your pallas kernel here")


# speedup vs baseline: 12.5132x; 12.5132x over previous
import jax
import jax.numpy as jnp
from jax.experimental import pallas as pl

H = 128
TI = 8


def _silu(x):
    return x * jax.nn.sigmoid(x)


def _vp_kernel(pos_ref, vp_ref):
    pos = pos_ref[:]
    n = pos.shape[0]
    center = jnp.mean(pos, axis=0, keepdims=True)
    v = pos - center
    vp_ref[:] = jnp.concatenate(
        [v, jnp.ones((n, 1), jnp.float32), jnp.zeros((n, H - 4), jnp.float32)],
        axis=1)


def _prep_kernel(h_ref, gf_ref, whh_ref, whg_ref, bh_ref, ws_ref, wr_ref,
                 h1_ref, a_ref, b_ref):
    f32 = jnp.float32
    row0 = jnp.dot(gf_ref[:], whg_ref[:], preferred_element_type=f32) + bh_ref[:]
    h1 = jnp.dot(h_ref[:], whh_ref[:], preferred_element_type=f32) + row0
    h1_ref[:] = h1
    a_ref[:] = jnp.dot(h1, ws_ref[:], preferred_element_type=f32)
    b_ref[:] = jnp.dot(h1, wr_ref[:], preferred_element_type=f32)


def _edge_kernel(vp_ref, vpt_ref, a_ref, bt_ref, wl_ref, b1_ref, w2_ref,
                 b2_ref, wt1_ref, bt1_ref, wt2_ref, bt2_ref, wx_ref, bx_ref,
                 we_ref, be_ref, msum_ref, aux_ref):
    f32 = jnp.float32
    k = pl.program_id(0)
    vp = vp_ref[:]
    vpt = vpt_ref[:]
    n = vp.shape[0]
    ti = vpt.shape[0]

    diff = vpt[:, None, :] - vp[None, :, :]
    l2 = jnp.sum(diff * diff, axis=2, keepdims=True)
    l3 = jnp.sqrt(jnp.maximum(l2, 1e-12))

    base = (bt_ref[:] + b1_ref[:])[:, None, :]
    pre1 = a_ref[:][None, :, :] + base + l3 * wl_ref[:][None, :, :]
    u = _silu(pre1).reshape(ti * n, H)
    m = _silu(jnp.dot(u, w2_ref[:], preferred_element_type=f32) + b2_ref[:])
    p = _silu(jnp.dot(m, wt1_ref[:], preferred_element_type=f32) + bt1_ref[:])
    p = _silu(jnp.dot(p, wt2_ref[:], preferred_element_type=f32) + bt2_ref[:])
    px = (jnp.dot(p, wx_ref[:], preferred_element_type=f32)
          + bx_ref[:]).reshape(ti, n, 1)
    ev = jax.nn.sigmoid(jnp.dot(m, we_ref[:], preferred_element_type=f32)
                        + be_ref[:]).reshape(ti, n, 1)

    i_idx = k * ti + jax.lax.broadcasted_iota(jnp.int32, (ti, n, 1), 0)
    j_idx = jax.lax.broadcasted_iota(jnp.int32, (ti, n, 1), 1)
    ev = jnp.where(i_idx == j_idx, 0.0, ev)

    m3 = m.reshape(ti, n, H)
    msum_ref[:] = jnp.sum(m3 * ev, axis=1)
    w3 = px / (1.0 + l3)
    aux_ref[:] = jnp.sum(w3 * vp[None, :, :], axis=1)


def _node_kernel(h1_ref, msum_ref, aux_ref, vp_ref, wp1m_ref, wp1h_ref,
                 bp1_ref, wp2_ref, bp2_ref, wp3_ref, bp3_ref, inv_ref,
                 h_ref, vpn_ref):
    f32 = jnp.float32
    inv_deg = inv_ref[0, 0]
    inv_sqrt_deg = inv_ref[0, 1]
    h1 = h1_ref[:]
    m_i = msum_ref[:] * inv_sqrt_deg
    t = _silu(jnp.dot(m_i, wp1m_ref[:], preferred_element_type=f32)
              + jnp.dot(h1, wp1h_ref[:], preferred_element_type=f32)
              + bp1_ref[:])
    t = _silu(jnp.dot(t, wp2_ref[:], preferred_element_type=f32) + bp2_ref[:])
    t = jnp.dot(t, wp3_ref[:], preferred_element_type=f32) + bp3_ref[:]
    h_ref[:] = t + h1

    aux = aux_ref[:]
    vp = vp_ref[:]
    col = jax.lax.broadcasted_iota(jnp.int32, aux.shape, 1)
    sw = jnp.sum(jnp.where(col == 3, aux, 0.0), axis=1, keepdims=True)
    vpn_ref[:] = vp + (sw * vp - aux) * inv_deg


def _final_kernel(vp_ref, posp_ref, h_ref, wf_ref, bf_ref, out_ref):
    scale = (jnp.dot(h_ref[:], wf_ref[:], preferred_element_type=jnp.float32)
             + bf_ref[:])
    out_ref[:] = (vp_ref[:] - posp_ref[:]) * scale


def kernel(positions, node_features, global_features, params):
    f32 = jnp.float32
    n = positions.shape[0]
    pos = positions.astype(f32)
    h = node_features.astype(f32)
    gf = global_features.reshape(1, -1).astype(f32)
    inv = jnp.array([[1.0 / (n - 1), 1.0 / jnp.sqrt(jnp.float32(n - 1)), 0.0,
                      0.0]], f32)

    vp = pl.pallas_call(
        _vp_kernel,
        out_shape=jax.ShapeDtypeStruct((n, H), f32),
    )(pos)

    for blk in params['blocks']:
        Wh, bh = blk['dense_h']
        (W1, b1), (W2, b2) = blk['phi_e']
        (Wt1, bt1), (Wt2, bt2) = blk['phi_x_torso']
        Wx, bx = blk['phi_x_final']
        We, be = blk['e_dense']
        (Wp1, bp1), (Wp2, bp2), (Wp3, bp3) = blk['phi_h']

        h1, A, B = pl.pallas_call(
            _prep_kernel,
            out_shape=[jax.ShapeDtypeStruct((n, H), f32)] * 3,
        )(h, gf, Wh[:H], Wh[H:], bh.reshape(1, -1), W1[:H], W1[H:2 * H])

        full = pl.BlockSpec((n, H), lambda i: (0, 0))
        wspec = lambda r: pl.BlockSpec((r, H), lambda i: (0, 0))
        colspec = lambda r: pl.BlockSpec((r, 1), lambda i: (0, 0))
        tiled = pl.BlockSpec((TI, H), lambda i: (i, 0))
        msum, aux = pl.pallas_call(
            _edge_kernel,
            grid=(n // TI,),
            in_specs=[full, tiled, full, tiled, wspec(1), wspec(1), wspec(H),
                      wspec(1), wspec(H), wspec(1), wspec(H), wspec(1),
                      colspec(H), colspec(1), colspec(H), colspec(1)],
            out_specs=[tiled, tiled],
            out_shape=[jax.ShapeDtypeStruct((n, H), f32)] * 2,
        )(vp, vp, A, B, W1[2 * H:], b1.reshape(1, -1), W2, b2.reshape(1, -1),
          Wt1, bt1.reshape(1, -1), Wt2, bt2.reshape(1, -1), Wx,
          bx.reshape(1, 1), We, be.reshape(1, 1))

        h, vp = pl.pallas_call(
            _node_kernel,
            out_shape=[jax.ShapeDtypeStruct((n, H), f32)] * 2,
        )(h1, msum, aux, vp, Wp1[:H], Wp1[H:], bp1.reshape(1, -1), Wp2,
          bp2.reshape(1, -1), Wp3, bp3.reshape(1, -1), inv)

    Wf, bf = params['final_dense']
    posp = jnp.pad(pos, ((0, 0), (0, H - 3)))
    out = pl.pallas_call(
        _final_kernel,
        out_shape=jax.ShapeDtypeStruct((n, H), f32),
    )(vp, posp, h, Wf, bf.reshape(1, 1))
    return out[:, :3]
